# baseline (speedup 1.0000x reference)
import jax
import jax.numpy as jnp
from jax.experimental import pallas as pl

NUM_CODES = 8192
EMBED_DIM = 256
COMMITMENT_COST = 0.25
M_BLK = 128


def _vq_body(z_ref, cb_ref, q_ref, loss_ref, idx_ref, perp_ref, cnt_ref):
    i = pl.program_id(0)
    nsteps = pl.num_programs(0)
    zb = z_ref[...]
    cb = cb_ref[...]
    mm = jax.lax.dot_general(
        zb, cb, (((1,), (1,)), ((), ())), preferred_element_type=jnp.float32
    )
    z2 = jnp.sum(zb * zb, axis=1, keepdims=True)
    c2 = jnp.sum(cb * cb, axis=1)
    dist = z2 - 2.0 * mm + c2[None, :]
    mv = jnp.min(dist, axis=1, keepdims=True)
    kiota = jax.lax.broadcasted_iota(jnp.int32, (M_BLK, NUM_CODES), 1)
    idx = jnp.min(
        jnp.where(dist == mv, kiota, jnp.int32(NUM_CODES)), axis=1
    ).astype(jnp.int32)
    oh = (kiota == idx[:, None]).astype(jnp.float32)
    q = jax.lax.dot_general(
        oh, cb, (((1,), (0,)), ((), ())), preferred_element_type=jnp.float32
    )
    q_ref[...] = zb + (q - zb)
    idx_ref[0, 0, :] = idx
    diff = q - zb
    psum = jnp.sum(diff * diff)
    pc = jnp.sum(oh, axis=0)

    @pl.when(i == 0)
    def _init():
        cnt_ref[...] = pc[None, :]
        loss_ref[...] = psum[None, None]

    @pl.when(i > 0)
    def _acc():
        cnt_ref[...] += pc[None, :]
        loss_ref[...] += psum[None, None]

    @pl.when(i == nsteps - 1)
    def _finish():
        n_tok = nsteps * M_BLK
        total = loss_ref[0, 0]
        loss_ref[...] = (COMMITMENT_COST * total / (n_tok * EMBED_DIM))[None, None]
        p = cnt_ref[...] * (1.0 / n_tok)
        ent = -jnp.sum(p * jnp.log(p + 1e-10))
        perp_ref[...] = jnp.exp(ent)[None, None]


def kernel(z, codebook):
    D = z.shape[-1]
    z_flat = z.reshape(-1, D)
    n_tok = z_flat.shape[0]
    grid = n_tok // M_BLK
    q, loss, idx3, perp, _counts = pl.pallas_call(
        _vq_body,
        grid=(grid,),
        in_specs=[
            pl.BlockSpec((M_BLK, D), lambda i: (i, 0)),
            pl.BlockSpec((NUM_CODES, D), lambda i: (0, 0)),
        ],
        out_specs=[
            pl.BlockSpec((M_BLK, D), lambda i: (i, 0)),
            pl.BlockSpec((1, 1), lambda i: (0, 0)),
            pl.BlockSpec((1, 1, M_BLK), lambda i: (i, 0, 0)),
            pl.BlockSpec((1, 1), lambda i: (0, 0)),
            pl.BlockSpec((1, NUM_CODES), lambda i: (0, 0)),
        ],
        out_shape=[
            jax.ShapeDtypeStruct((n_tok, D), jnp.float32),
            jax.ShapeDtypeStruct((1, 1), jnp.float32),
            jax.ShapeDtypeStruct((grid, 1, M_BLK), jnp.int32),
            jax.ShapeDtypeStruct((1, 1), jnp.float32),
            jax.ShapeDtypeStruct((1, NUM_CODES), jnp.float32),
        ],
    )(z_flat, codebook)
    return (
        q.reshape(z.shape),
        loss[0, 0],
        idx3.reshape(z.shape[:-1]),
        perp[0, 0],
    )
